# baseline (speedup 1.0000x reference)
import functools

import jax
import jax.numpy as jnp
from jax import lax
from jax.experimental import pallas as pl
from jax.experimental.pallas import tpu as pltpu
from jax.experimental.pallas import tpu_sc as plsc

N = 10000
E = 320000
DH = 128
DZ = 32
NC = 2
NS = 16
ACC_N = 10240
RPT = ACC_N // NS
EPT = E // (NC * NS)
K = 80
NCHUNK = EPT // K
ZR = 128

_mesh = plsc.VectorSubcoreMesh(
    core_axis_name="c", subcore_axis_name="s", num_cores=NC, num_subcores=NS
)


def _vcopy_chunk(src_all, off, dstbuf):
    for i in range(K // 16):
        dstbuf[pl.ds(i * 16, 16)] = src_all[pl.ds(off + i * 16, 16)]


@functools.partial(
    pl.kernel,
    out_type=jax.ShapeDtypeStruct((NC, ACC_N), jnp.float32),
    mesh=_mesh,
    scratch_types=[
        pltpu.VMEM((EPT,), jnp.int32),
        pltpu.VMEM((4, K), jnp.int32),
        pltpu.VMEM((K,), jnp.float32),
        pltpu.VMEM((RPT,), jnp.float32),
        pltpu.VMEM_SHARED((ACC_N,), jnp.float32),
        pltpu.SemaphoreType.DMA,
        pltpu.SemaphoreType.DMA,
    ],
)
def _deg_kernel(dst_hbm, out_hbm, didx_all, dbuf, ones, zbuf, acc, isem, ssem):
    c = lax.axis_index("c")
    s = lax.axis_index("s")
    ebase = (c * NS + s) * EPT
    pltpu.async_copy(dst_hbm.at[pl.ds(ebase, EPT)], didx_all, isem)

    zero16 = jnp.zeros((16,), jnp.float32)
    one16 = jnp.ones((16,), jnp.float32)

    def _init(i, carry):
        zbuf[pl.ds(i * 16, 16)] = zero16
        return carry

    lax.fori_loop(0, RPT // 16, _init, 0)
    for j in range(K // 16):
        ones[pl.ds(j * 16, 16)] = one16

    pltpu.sync_copy(zbuf, acc.at[pl.ds(s * RPT, RPT)])
    pltpu.make_async_copy(dst_hbm.at[pl.ds(ebase, EPT)], didx_all, isem).wait()
    plsc.subcore_barrier()

    for q in range(4):
        _vcopy_chunk(didx_all, q * K, dbuf.at[q])
        pltpu.async_copy(ones, acc.at[dbuf.at[q]], ssem, add=True)

    def _quad(j, carry):
        i0 = 4 * j + 4
        for q in range(4):
            pltpu.make_async_copy(ones, acc.at[dbuf.at[q]], ssem).wait()
            _vcopy_chunk(didx_all, (i0 + q) * K, dbuf.at[q])
            pltpu.async_copy(ones, acc.at[dbuf.at[q]], ssem, add=True)
        return carry

    lax.fori_loop(0, (NCHUNK - 1) // 4 - 1, _quad, 0)
    for q in range(4):
        pltpu.make_async_copy(ones, acc.at[dbuf.at[q]], ssem).wait()
    _vcopy_chunk(didx_all, (NCHUNK - 1) * K, dbuf.at[0])
    pltpu.sync_copy(ones, acc.at[dbuf.at[0]], add=True)

    plsc.subcore_barrier()
    pltpu.sync_copy(acc.at[pl.ds(s * RPT, RPT)], out_hbm.at[c, pl.ds(s * RPT, RPT)])


@functools.partial(
    pl.kernel,
    out_type=jax.ShapeDtypeStruct((NC, ACC_N, DH), jnp.float32),
    mesh=_mesh,
    scratch_types=[
        pltpu.VMEM((EPT,), jnp.int32),
        pltpu.VMEM((NCHUNK, K), jnp.int32),
        pltpu.VMEM((2, K, DH), jnp.float32),
        pltpu.VMEM_SHARED((ACC_N, DH), jnp.float32),
        pltpu.SemaphoreType.DMA,
        pltpu.SemaphoreType.DMA,
    ],
)
def _scatter_kernel(y_hbm, src_hbm, dst3_hbm, out_hbm, sidx_all, didx_all,
                    rows, acc, isem, gsem):
    c = lax.axis_index("c")
    s = lax.axis_index("s")
    tile = c * NS + s
    ebase = tile * EPT
    pltpu.async_copy(src_hbm.at[pl.ds(ebase, EPT)], sidx_all, isem)
    pltpu.async_copy(dst3_hbm.at[tile], didx_all, isem)

    zero16 = jnp.zeros((16,), jnp.float32)

    def _zrow(i, carry):
        for j in range(DH // 16):
            rows[0, i, pl.ds(j * 16, 16)] = zero16
            rows[1, i, pl.ds(j * 16, 16)] = zero16
        return carry

    lax.fori_loop(0, K, _zrow, 0)
    for t in range(RPT // K):
        pltpu.sync_copy(rows.at[t % 2], acc.at[pl.ds(s * RPT + t * K, K)])
    pltpu.make_async_copy(src_hbm.at[pl.ds(ebase, EPT)], sidx_all, isem).wait()
    pltpu.make_async_copy(dst3_hbm.at[tile], didx_all, isem).wait()
    plsc.subcore_barrier()

    def _gather(i, buf):
        pltpu.async_copy(y_hbm.at[sidx_all.at[pl.ds(i * K, K)]], rows.at[buf],
                         gsem)

    def _gwait(i, buf):
        pltpu.make_async_copy(y_hbm.at[sidx_all.at[pl.ds(i * K, K)]],
                              rows.at[buf], gsem).wait()

    _gather(0, 0)

    def _pair(j, carry):
        i0 = 2 * j
        _gwait(i0, 0)
        _gather(i0 + 1, 1)
        pltpu.sync_copy(rows.at[0], acc.at[didx_all.at[i0]], add=True)
        _gwait(i0 + 1, 1)
        _gather(i0 + 2, 0)
        pltpu.sync_copy(rows.at[1], acc.at[didx_all.at[i0 + 1]], add=True)
        return carry

    lax.fori_loop(0, (NCHUNK - 1) // 2, _pair, 0)
    _gwait(NCHUNK - 1, 0)
    pltpu.sync_copy(rows.at[0], acc.at[didx_all.at[NCHUNK - 1]], add=True)
    plsc.subcore_barrier()
    pltpu.sync_copy(acc.at[pl.ds(s * RPT, RPT)], out_hbm.at[c, pl.ds(s * RPT, RPT)])


def _tc1_body(x_ref, w1_ref, dp0_ref, dp1_ref, y_ref, dinv_ref):
    deg = dp0_ref[pl.ds(0, N), :] + dp1_ref[pl.ds(0, N), :] + 1.0
    dinv = lax.rsqrt(deg)
    xw = jnp.dot(x_ref[...], w1_ref[...], preferred_element_type=jnp.float32)
    y_ref[...] = xw * dinv
    dinv_ref[...] = dinv


_tc1 = pl.pallas_call(
    _tc1_body,
    out_shape=[
        jax.ShapeDtypeStruct((N, DH), jnp.float32),
        jax.ShapeDtypeStruct((N, 1), jnp.float32),
    ],
)


def _tc2_body(s_ref, y_ref, dinv_ref, b1_ref, w2_ref, y2_ref):
    dinv = dinv_ref[...]
    ssum = s_ref[0, pl.ds(0, N), :] + s_ref[1, pl.ds(0, N), :]
    h = jnp.maximum(dinv * (ssum + y_ref[...]) + b1_ref[...], 0.0)
    y2_ref[...] = jnp.dot(h, w2_ref[...], preferred_element_type=jnp.float32) * dinv


_tc2 = pl.pallas_call(
    _tc2_body,
    out_shape=jax.ShapeDtypeStruct((N, DH), jnp.float32),
)


def _tc3_body(s_ref, y2_ref, dinv_ref, b2_ref, wmu_ref, bmu_ref, wlv_ref, blv_ref,
              mu_ref, lv_ref):
    dinv = dinv_ref[...]
    ssum = s_ref[0, pl.ds(0, N), :] + s_ref[1, pl.ds(0, N), :]
    h = jnp.maximum(dinv * (ssum + y2_ref[...]) + b2_ref[...], 0.0)
    mu_ref[...] = jnp.dot(h, wmu_ref[...], preferred_element_type=jnp.float32) + bmu_ref[...]
    lv_ref[...] = jnp.dot(h, wlv_ref[...], preferred_element_type=jnp.float32) + blv_ref[...]


_tc3 = pl.pallas_call(
    _tc3_body,
    out_shape=[
        jax.ShapeDtypeStruct((N, DZ), jnp.float32),
        jax.ShapeDtypeStruct((N, DZ), jnp.float32),
    ],
)


def kernel(x, edge_index, W1, b1, W2, b2, Wmu, bmu, Wlv, blv):
    srcc = edge_index[0]
    dstc = edge_index[1]
    dst3 = dstc.reshape(NC * NS, NCHUNK, K)
    degp = _deg_kernel(dstc)
    dp0 = degp[0].reshape(ACC_N, 1)
    dp1 = degp[1].reshape(ACC_N, 1)
    y1, dinv = _tc1(x, W1, dp0, dp1)
    s1 = _scatter_kernel(y1, srcc, dst3)
    y2 = _tc2(s1, y1, dinv, b1.reshape(1, DH), W2)
    s2 = _scatter_kernel(y2, srcc, dst3)
    mu, lv = _tc3(s2, y2, dinv, b2.reshape(1, DH), Wmu, bmu.reshape(1, DZ), Wlv, blv.reshape(1, DZ))
    return (mu, lv)


# speedup vs baseline: 1.4449x; 1.4449x over previous
import functools

import jax
import jax.numpy as jnp
from jax import lax
from jax.experimental import pallas as pl
from jax.experimental.pallas import tpu as pltpu
from jax.experimental.pallas import tpu_sc as plsc

N = 10000
E = 320000
DH = 128
DZ = 32
NC = 2
NS = 16
ACC_N = 10240
RPT = ACC_N // NS
EPT = E // (NC * NS)
K = 80
NCHUNK = EPT // K
ZR = 128

_mesh = plsc.VectorSubcoreMesh(
    core_axis_name="c", subcore_axis_name="s", num_cores=NC, num_subcores=NS
)


def _vcopy_chunk(src_all, off, dstbuf):
    for i in range(K // 16):
        dstbuf[pl.ds(i * 16, 16)] = src_all[pl.ds(off + i * 16, 16)]


@functools.partial(
    pl.kernel,
    out_type=jax.ShapeDtypeStruct((NC, ACC_N), jnp.float32),
    mesh=_mesh,
    scratch_types=[
        pltpu.VMEM((EPT,), jnp.int32),
        pltpu.VMEM((4, K), jnp.int32),
        pltpu.VMEM((K,), jnp.float32),
        pltpu.VMEM((RPT,), jnp.float32),
        pltpu.VMEM_SHARED((ACC_N,), jnp.float32),
        pltpu.SemaphoreType.DMA,
        pltpu.SemaphoreType.DMA,
    ],
)
def _deg_kernel(dst_hbm, out_hbm, didx_all, dbuf, ones, zbuf, acc, isem, ssem):
    c = lax.axis_index("c")
    s = lax.axis_index("s")
    ebase = (c * NS + s) * EPT
    pltpu.async_copy(dst_hbm.at[pl.ds(ebase, EPT)], didx_all, isem)

    zero16 = jnp.zeros((16,), jnp.float32)
    one16 = jnp.ones((16,), jnp.float32)

    def _init(i, carry):
        zbuf[pl.ds(i * 16, 16)] = zero16
        return carry

    lax.fori_loop(0, RPT // 16, _init, 0)
    for j in range(K // 16):
        ones[pl.ds(j * 16, 16)] = one16

    pltpu.sync_copy(zbuf, acc.at[pl.ds(s * RPT, RPT)])
    pltpu.make_async_copy(dst_hbm.at[pl.ds(ebase, EPT)], didx_all, isem).wait()
    plsc.subcore_barrier()

    for q in range(4):
        _vcopy_chunk(didx_all, q * K, dbuf.at[q])
        pltpu.async_copy(ones, acc.at[dbuf.at[q]], ssem, add=True)

    def _quad(j, carry):
        i0 = 4 * j + 4
        for q in range(4):
            pltpu.make_async_copy(ones, acc.at[dbuf.at[q]], ssem).wait()
            _vcopy_chunk(didx_all, (i0 + q) * K, dbuf.at[q])
            pltpu.async_copy(ones, acc.at[dbuf.at[q]], ssem, add=True)
        return carry

    lax.fori_loop(0, (NCHUNK - 1) // 4 - 1, _quad, 0)
    for q in range(4):
        pltpu.make_async_copy(ones, acc.at[dbuf.at[q]], ssem).wait()
    _vcopy_chunk(didx_all, (NCHUNK - 1) * K, dbuf.at[0])
    pltpu.sync_copy(ones, acc.at[dbuf.at[0]], add=True)

    plsc.subcore_barrier()
    pltpu.sync_copy(acc.at[pl.ds(s * RPT, RPT)], out_hbm.at[c, pl.ds(s * RPT, RPT)])


@functools.partial(
    pl.kernel,
    out_type=jax.ShapeDtypeStruct((NC, ACC_N, DH), jnp.float32),
    mesh=_mesh,
    scratch_types=[
        pltpu.VMEM((EPT,), jnp.int32),
        pltpu.VMEM((3, K), jnp.int32),
        pltpu.VMEM((3, K, DH), jnp.float32),
        pltpu.VMEM_SHARED((ACC_N, DH), jnp.float32),
        pltpu.SemaphoreType.DMA,
        pltpu.SemaphoreType.DMA,
        pltpu.SemaphoreType.DMA,
    ],
)
def _scatter_kernel(y_hbm, src_hbm, dst_hbm, out_hbm, sidx_all, dbuf,
                    rows, acc, isem, dsem, gsem):
    c = lax.axis_index("c")
    s = lax.axis_index("s")
    tile = c * NS + s
    ebase = tile * EPT
    pltpu.async_copy(src_hbm.at[pl.ds(ebase, EPT)], sidx_all, isem)

    zero16 = jnp.zeros((16,), jnp.float32)

    def _zrow(i, carry):
        for j in range(DH // 16):
            rows[0, i, pl.ds(j * 16, 16)] = zero16
            rows[1, i, pl.ds(j * 16, 16)] = zero16
        return carry

    lax.fori_loop(0, K, _zrow, 0)
    for t in range(RPT // K):
        pltpu.sync_copy(rows.at[t % 2], acc.at[pl.ds(s * RPT + t * K, K)])
    pltpu.make_async_copy(src_hbm.at[pl.ds(ebase, EPT)], sidx_all, isem).wait()
    plsc.subcore_barrier()

    def _dload(i, buf):
        pltpu.async_copy(dst_hbm.at[pl.ds(ebase + i * K, K)], dbuf.at[buf],
                         dsem)

    def _dwait(i, buf):
        pltpu.make_async_copy(dst_hbm.at[pl.ds(ebase + i * K, K)],
                              dbuf.at[buf], dsem).wait()

    def _gather(i, buf):
        pltpu.async_copy(y_hbm.at[sidx_all.at[pl.ds(i * K, K)]], rows.at[buf],
                         gsem)

    def _gwait(i, buf):
        pltpu.make_async_copy(y_hbm.at[sidx_all.at[pl.ds(i * K, K)]],
                              rows.at[buf], gsem).wait()

    for q in range(3):
        _dload(q, q)
        _gather(q, q)

    def _step(i, buf, more):
        _gwait(i, buf)
        _dwait(i, buf)
        pltpu.sync_copy(rows.at[buf], acc.at[dbuf.at[buf]], add=True)
        if more:
            _dload(i + 3, buf)
            _gather(i + 3, buf)

    def _tri(j, carry):
        i0 = 3 * j
        _step(i0, 0, True)
        _step(i0 + 1, 1, True)
        _step(i0 + 2, 2, True)
        return carry

    lax.fori_loop(0, (NCHUNK - 5) // 3, _tri, 0)
    _step(NCHUNK - 5, 0, True)
    _step(NCHUNK - 4, 1, True)
    _step(NCHUNK - 3, 2, False)
    _step(NCHUNK - 2, 0, False)
    _step(NCHUNK - 1, 1, False)
    plsc.subcore_barrier()
    pltpu.sync_copy(acc.at[pl.ds(s * RPT, RPT)], out_hbm.at[c, pl.ds(s * RPT, RPT)])


def _tc1_body(x_ref, w1_ref, dp0_ref, dp1_ref, y_ref, dinv_ref):
    deg = dp0_ref[pl.ds(0, N), :] + dp1_ref[pl.ds(0, N), :] + 1.0
    dinv = lax.rsqrt(deg)
    xw = jnp.dot(x_ref[...], w1_ref[...], preferred_element_type=jnp.float32)
    y_ref[...] = xw * dinv
    dinv_ref[...] = dinv


_tc1 = pl.pallas_call(
    _tc1_body,
    out_shape=[
        jax.ShapeDtypeStruct((N, DH), jnp.float32),
        jax.ShapeDtypeStruct((N, 1), jnp.float32),
    ],
)


def _tc2_body(s_ref, y_ref, dinv_ref, b1_ref, w2_ref, y2_ref):
    dinv = dinv_ref[...]
    ssum = s_ref[0, pl.ds(0, N), :] + s_ref[1, pl.ds(0, N), :]
    h = jnp.maximum(dinv * (ssum + y_ref[...]) + b1_ref[...], 0.0)
    y2_ref[...] = jnp.dot(h, w2_ref[...], preferred_element_type=jnp.float32) * dinv


_tc2 = pl.pallas_call(
    _tc2_body,
    out_shape=jax.ShapeDtypeStruct((N, DH), jnp.float32),
)


def _tc3_body(s_ref, y2_ref, dinv_ref, b2_ref, wmu_ref, bmu_ref, wlv_ref, blv_ref,
              mu_ref, lv_ref):
    dinv = dinv_ref[...]
    ssum = s_ref[0, pl.ds(0, N), :] + s_ref[1, pl.ds(0, N), :]
    h = jnp.maximum(dinv * (ssum + y2_ref[...]) + b2_ref[...], 0.0)
    mu_ref[...] = jnp.dot(h, wmu_ref[...], preferred_element_type=jnp.float32) + bmu_ref[...]
    lv_ref[...] = jnp.dot(h, wlv_ref[...], preferred_element_type=jnp.float32) + blv_ref[...]


_tc3 = pl.pallas_call(
    _tc3_body,
    out_shape=[
        jax.ShapeDtypeStruct((N, DZ), jnp.float32),
        jax.ShapeDtypeStruct((N, DZ), jnp.float32),
    ],
)


def kernel(x, edge_index, W1, b1, W2, b2, Wmu, bmu, Wlv, blv):
    srcc = edge_index[0]
    dstc = edge_index[1]
    degp = _deg_kernel(dstc)
    dp0 = degp[0].reshape(ACC_N, 1)
    dp1 = degp[1].reshape(ACC_N, 1)
    y1, dinv = _tc1(x, W1, dp0, dp1)
    s1 = _scatter_kernel(y1, srcc, dstc)
    y2 = _tc2(s1, y1, dinv, b1.reshape(1, DH), W2)
    s2 = _scatter_kernel(y2, srcc, dstc)
    mu, lv = _tc3(s2, y2, dinv, b2.reshape(1, DH), Wmu, bmu.reshape(1, DZ), Wlv, blv.reshape(1, DZ))
    return (mu, lv)
